# baseline (speedup 1.0000x reference)
import functools

import jax
import jax.numpy as jnp
from jax import lax
from jax.experimental import pallas as pl
from jax.experimental.pallas import tpu as pltpu
from jax.experimental.pallas import tpu_sc as plsc

BATCH = 16384
EMB = 32
L = 16
NC = 2
NS = 16
NW = NC * NS
BPW = BATCH // NW
GCHUNK = 128


def _ncf_sc_kernel(uidx_hbm, iidx_hbm, user_table, ub_table, item_table,
                   ib_table, wf_hbm, bf_hbm, out_hbm,
                   uidx_v, iidx_v, urows, irows, ub_v, ib_v, wf_v, bf_v,
                   out_v, sem):
    wid = lax.axis_index("s") * NC + lax.axis_index("c")
    base = wid * BPW

    pltpu.sync_copy(uidx_hbm.at[pl.ds(base, BPW)], uidx_v)
    pltpu.sync_copy(iidx_hbm.at[pl.ds(base, BPW)], iidx_v)
    pltpu.sync_copy(wf_hbm, wf_v)
    pltpu.sync_copy(bf_hbm, bf_v)

    copies = []
    for j in range(BPW // GCHUNK):
        sl = pl.ds(j * GCHUNK, GCHUNK)
        copies.append(pltpu.async_copy(user_table.at[uidx_v.at[sl]],
                                       urows.at[sl], sem))
        copies.append(pltpu.async_copy(item_table.at[iidx_v.at[sl]],
                                       irows.at[sl], sem))
        copies.append(pltpu.async_copy(ub_table.at[uidx_v.at[sl]],
                                       ub_v.at[sl], sem))
        copies.append(pltpu.async_copy(ib_table.at[iidx_v.at[sl]],
                                       ib_v.at[sl], sem))
    for c in copies:
        c.wait()

    wf = wf_v[...]
    bf = bf_v[...]
    lane = lax.iota(jnp.int32, L)

    def group(g, carry):
        acc = jnp.zeros((L,), jnp.float32)
        for r in range(L):
            row = g * L + r
            u0 = urows[row, pl.ds(0, L)]
            u1 = urows[row, pl.ds(L, L)]
            v0 = irows[row, pl.ds(0, L)]
            v1 = irows[row, pl.ds(L, L)]
            s = u0 * v0 + u1 * v1
            acc = jnp.where(lane == r, jnp.sum(s), acc)
        sl = pl.ds(g * L, L)
        acc = acc + ub_v[sl] + ib_v[sl]
        t = acc * wf + bf
        out_v[sl] = 1.0 / (1.0 + jnp.exp(-t))
        return carry

    lax.fori_loop(0, BPW // L, group, 0)
    pltpu.sync_copy(out_v, out_hbm.at[pl.ds(base, BPW)])


@jax.jit
def _ncf_forward(uidx, iidx, user_table, ub_flat, item_table, ib_flat,
                 wf_vec, bf_vec):
    mesh = plsc.VectorSubcoreMesh(core_axis_name="c", subcore_axis_name="s")
    run = pl.kernel(
        _ncf_sc_kernel,
        mesh=mesh,
        compiler_params=pltpu.CompilerParams(needs_layout_passes=False,
                                             use_tc_tiling_on_sc=False),
        out_type=jax.ShapeDtypeStruct((BATCH,), jnp.float32),
        scratch_types=[
            pltpu.VMEM((BPW,), jnp.int32),
            pltpu.VMEM((BPW,), jnp.int32),
            pltpu.VMEM((BPW, EMB), jnp.float32),
            pltpu.VMEM((BPW, EMB), jnp.float32),
            pltpu.VMEM((BPW,), jnp.float32),
            pltpu.VMEM((BPW,), jnp.float32),
            pltpu.VMEM((L,), jnp.float32),
            pltpu.VMEM((L,), jnp.float32),
            pltpu.VMEM((BPW,), jnp.float32),
            pltpu.SemaphoreType.DMA,
        ],
    )
    return run(uidx, iidx, user_table, ub_flat, item_table, ib_flat,
               wf_vec, bf_vec)


def kernel(inputs, user_table, user_bias_table, item_table, item_bias_table,
           W1, b1, W2, b2, W3, b3, Wf, bf):
    del W1, b1, W2, b2, W3, b3
    uidx = inputs[:, 0].astype(jnp.int32)
    iidx = inputs[:, 1].astype(jnp.int32)
    wf_vec = jnp.broadcast_to(Wf.reshape(()), (L,)).astype(jnp.float32)
    bf_vec = jnp.broadcast_to(bf.reshape(()), (L,)).astype(jnp.float32)
    out = _ncf_forward(uidx, iidx, user_table,
                       user_bias_table.reshape(-1), item_table,
                       item_bias_table.reshape(-1), wf_vec, bf_vec)
    return out.reshape(BATCH, 1)
